# baseline (speedup 1.0000x reference)
import jax
import jax.numpy as jnp
from jax.experimental import pallas as pl
from jax.experimental.pallas import tpu as pltpu

TIME = 4096
FREQ = 513
CHUNK = 20
N_WIN = TIME - CHUNK
JB = 128
NJ = (N_WIN + JB - 1) // JB
PADW = TIME + 128


def _body(mt_hbm, vt_hbm, x_ref, y_ref, mscr, vscr, sem0, sem1):
    jb = pl.program_id(0)

    @pl.when(jb == 0)
    def _():
        cp0 = pltpu.make_async_copy(mt_hbm, mscr.at[:, pl.ds(0, TIME)], sem0)
        cp1 = pltpu.make_async_copy(vt_hbm, vscr.at[:, pl.ds(0, TIME)], sem1)
        cp0.start()
        cp1.start()
        cp0.wait()
        cp1.wait()

    base = pl.multiple_of(jb * JB, 128)
    w = mscr[:, pl.ds(base, JB + 128)]
    for c in range(CHUNK):
        x_ref[c, :, :] = w[:, c:c + JB]
    wv = vscr[:, pl.ds(base, JB + 128)]
    y_ref[...] = wv[:, CHUNK:CHUNK + JB]


_call = pl.pallas_call(
    _body,
    grid=(NJ,),
    in_specs=[
        pl.BlockSpec(memory_space=pl.ANY),
        pl.BlockSpec(memory_space=pl.ANY),
    ],
    out_specs=[
        pl.BlockSpec((CHUNK, FREQ, JB), lambda j: (0, 0, j)),
        pl.BlockSpec((FREQ, JB), lambda j: (0, j)),
    ],
    out_shape=[
        jax.ShapeDtypeStruct((CHUNK, FREQ, N_WIN), jnp.float32),
        jax.ShapeDtypeStruct((FREQ, N_WIN), jnp.float32),
    ],
    scratch_shapes=[
        pltpu.VMEM((FREQ, PADW), jnp.float32),
        pltpu.VMEM((FREQ, PADW), jnp.float32),
        pltpu.SemaphoreType.DMA,
        pltpu.SemaphoreType.DMA,
    ],
    compiler_params=pltpu.CompilerParams(vmem_limit_bytes=58 * 1024 * 1024),
)


def kernel(mixed_mag, vocal_mag):
    mt = mixed_mag.T
    vt = vocal_mag.T
    x_alt, y_alt = _call(mt, vt)
    return x_alt.transpose(2, 1, 0), y_alt.T


# speedup vs baseline: 1.3465x; 1.3465x over previous
import jax
import jax.numpy as jnp
from jax.experimental import pallas as pl
from jax.experimental.pallas import tpu as pltpu

TIME = 4096
FREQ = 513
CHUNK = 20
N_WIN = TIME - CHUNK
JB = 384
NJ = (N_WIN + JB - 1) // JB
PADW = TIME + 128


def _body(mt_hbm, vt_hbm, x_ref, y_ref, mscr, vscr, sem0, sem1):
    jb = pl.program_id(0)

    @pl.when(jb == 0)
    def _():
        cp0 = pltpu.make_async_copy(mt_hbm, mscr.at[:, pl.ds(0, TIME)], sem0)
        cp1 = pltpu.make_async_copy(vt_hbm, vscr.at[:, pl.ds(0, TIME)], sem1)
        cp0.start()
        cp1.start()
        cp0.wait()
        cp1.wait()

    base = pl.multiple_of(jb * JB, 128)
    w = mscr[:, pl.ds(base, JB + 128)]
    for c in range(CHUNK):
        x_ref[c, :, :] = w[:, c:c + JB]
    wv = vscr[:, pl.ds(base, JB + 128)]
    y_ref[...] = wv[:, CHUNK:CHUNK + JB]


_call = pl.pallas_call(
    _body,
    grid=(NJ,),
    in_specs=[
        pl.BlockSpec(memory_space=pl.ANY),
        pl.BlockSpec(memory_space=pl.ANY),
    ],
    out_specs=[
        pl.BlockSpec((CHUNK, FREQ, JB), lambda j: (0, 0, j)),
        pl.BlockSpec((FREQ, JB), lambda j: (0, j)),
    ],
    out_shape=[
        jax.ShapeDtypeStruct((CHUNK, FREQ, N_WIN), jnp.float32),
        jax.ShapeDtypeStruct((FREQ, N_WIN), jnp.float32),
    ],
    scratch_shapes=[
        pltpu.VMEM((FREQ, PADW), jnp.float32),
        pltpu.VMEM((FREQ, PADW), jnp.float32),
        pltpu.SemaphoreType.DMA,
        pltpu.SemaphoreType.DMA,
    ],
    compiler_params=pltpu.CompilerParams(vmem_limit_bytes=58 * 1024 * 1024),
)


def kernel(mixed_mag, vocal_mag):
    mt = mixed_mag.T
    vt = vocal_mag.T
    x_alt, y_alt = _call(mt, vt)
    return x_alt.transpose(2, 1, 0), y_alt.T
